# Initial kernel scaffold; baseline (speedup 1.0000x reference)
import jax
import jax.numpy as jnp
from jax.experimental import pallas as pl


def kernel(x, edge_idx, edge_attr, g, mu, sigma, W_root, bias, gamma, beta):
    raise NotImplementedError("write your pallas kernel here")


# speedup vs baseline: 3.6948x; 3.6948x over previous
import functools

import jax
import jax.numpy as jnp
from jax import lax
from jax.experimental import pallas as pl
from jax.experimental.pallas import tpu as pltpu
from jax.experimental.pallas import tpu_sc as plsc

N = 10000
E = 320000
D_IN = 128
D_OUT = 128
K = 4
DIM = 16
EPS = 1e-15

NUM_CORES = 2
NUM_SUBCORES = 16
NUM_TILES = NUM_CORES * NUM_SUBCORES
E_PER_TILE = E // NUM_TILES
CHUNK = 40
NUM_CHUNKS = E_PER_TILE // CHUNK
ACC_W = 128
N_PAD = 10240
CNT_BLK = 512
CNT_STEPS = E // CNT_BLK
ROWS_PER_TILE = N_PAD // NUM_SUBCORES


def _mm_body(x_ref, g_ref, wr_ref, xg_ref, xr_ref):
  x = x_ref[...]
  xg_ref[...] = jnp.dot(x, g_ref[...], preferred_element_type=jnp.float32)
  xr_ref[...] = jnp.dot(x, wr_ref[...], preferred_element_type=jnp.float32)


def _input_transform(x, g, w_root):
  return pl.pallas_call(
      _mm_body,
      out_shape=(
          jax.ShapeDtypeStruct((N, K * D_OUT), jnp.float32),
          jax.ShapeDtypeStruct((N, D_OUT), jnp.float32),
      ),
  )(x, g, w_root)


def _sc_body(xg_hbm, src_hbm, dst_hbm, ea_hbm, mu_hbm, sig_hbm, zeros_hbm,
             out_hbm, acc, src_v, dst_v, ea_v, rows_v, msg_v, mu_v, sig_v,
             sem):
  cid = lax.axis_index("c")
  sid = lax.axis_index("s")
  wid = sid * NUM_CORES + cid

  row0 = sid * ROWS_PER_TILE
  pltpu.sync_copy(zeros_hbm.at[pl.ds(row0, ROWS_PER_TILE)],
                  acc.at[pl.ds(row0, ROWS_PER_TILE)])

  pltpu.sync_copy(mu_hbm, mu_v)
  pltpu.sync_copy(sig_hbm, sig_v)
  mus = []
  invs = []
  for k in range(K):
    m = mu_v[k]
    s = sig_v[k]
    mus.append(m)
    invs.append(0.5 / (EPS + s * s))

  lanes = lax.iota(jnp.int32, 16)
  bfly = [lax.bitwise_xor(lanes, jnp.full((16,), sh, jnp.int32))
          for sh in (8, 4, 2, 1)]

  dnums = lax.GatherDimensionNumbers(
      offset_dims=(), collapsed_slice_dims=(0,), start_index_map=(0,))

  def lane_sum(v):
    for idx in bfly:
      v = v + lax.gather(v, idx[:, None], dnums, (1,),
                         mode=lax.GatherScatterMode.PROMISE_IN_BOUNDS)
    return v

  plsc.subcore_barrier()

  base_tile = wid * E_PER_TILE

  def chunk_body(gi, _):
    base = base_tile + gi * CHUNK
    pltpu.sync_copy(src_hbm.at[pl.ds(base, CHUNK)], src_v)
    pltpu.sync_copy(dst_hbm.at[pl.ds(base, CHUNK)], dst_v)
    pltpu.sync_copy(ea_hbm.at[pl.ds(base, CHUNK)], ea_v)
    pltpu.async_copy(xg_hbm.at[src_v], rows_v, sem).wait()

    def edge_body(e, _):
      ea = ea_v[e]
      accs = [None] * (D_OUT // 16)
      for k in range(K):
        d = ea - mus[k]
        t = d * d * invs[k]
        w = jnp.exp(-lane_sum(t))
        for j in range(D_OUT // 16):
          r = rows_v[e, pl.ds(k * D_OUT + j * 16, 16)]
          if k == 0:
            accs[j] = w * r
          else:
            accs[j] = accs[j] + w * r
      for j in range(D_OUT // 16):
        msg_v[e, pl.ds(j * 16, 16)] = accs[j]
      return 0

    lax.fori_loop(0, CHUNK, edge_body, 0)

    pltpu.sync_copy(msg_v, acc.at[dst_v], add=True)
    return 0

  lax.fori_loop(0, NUM_CHUNKS, chunk_body, 0)

  plsc.subcore_barrier()

  pltpu.sync_copy(acc.at[pl.ds(row0, ROWS_PER_TILE)],
                  out_hbm.at[cid, pl.ds(row0, ROWS_PER_TILE)])


def _sc_aggregate(xg, src, dst, edge_attr, mu, sigma, zeros):
  mesh = plsc.VectorSubcoreMesh(
      core_axis_name="c", subcore_axis_name="s",
      num_cores=NUM_CORES, num_subcores=NUM_SUBCORES)
  run = pl.kernel(
      _sc_body,
      out_type=jax.ShapeDtypeStruct((NUM_CORES, N_PAD, ACC_W), jnp.float32),
      mesh=mesh,
      scratch_types=[
          pltpu.VMEM_SHARED((N_PAD, ACC_W), jnp.float32),
          pltpu.VMEM((CHUNK,), jnp.int32),
          pltpu.VMEM((CHUNK,), jnp.int32),
          pltpu.VMEM((CHUNK, DIM), jnp.float32),
          pltpu.VMEM((CHUNK, K * D_OUT), jnp.float32),
          pltpu.VMEM((CHUNK, ACC_W), jnp.float32),
          pltpu.VMEM((K, DIM), jnp.float32),
          pltpu.VMEM((K, DIM), jnp.float32),
          pltpu.SemaphoreType.DMA,
      ],
  )
  return run(xg, src, dst, edge_attr, mu, sigma, zeros)


def _cnt_body(dr_ref, dc_ref, o_ref):
  i = pl.program_id(0)
  d_row = dr_ref[0]
  d_col = dc_ref[0]
  hi_row = d_row // 128
  lo_col = d_col - (d_col // 128) * 128
  ha = (lax.broadcasted_iota(jnp.int32, (N_PAD // 128, CNT_BLK), 0)
        == hi_row).astype(jnp.float32)
  hb = (lax.broadcasted_iota(jnp.int32, (CNT_BLK, 128), 1)
        == lo_col).astype(jnp.float32)

  @pl.when(i == 0)
  def _():
    o_ref[...] = jnp.zeros_like(o_ref)

  o_ref[...] += jnp.dot(ha, hb, preferred_element_type=jnp.float32)


def _degree_count(dst):
  d3r = dst.reshape(CNT_STEPS, 1, CNT_BLK)
  d3c = dst.reshape(CNT_STEPS, CNT_BLK, 1)
  return pl.pallas_call(
      _cnt_body,
      grid=(CNT_STEPS,),
      in_specs=[
          pl.BlockSpec((1, 1, CNT_BLK), lambda i: (i, 0, 0)),
          pl.BlockSpec((1, CNT_BLK, 1), lambda i: (i, 0, 0)),
      ],
      out_specs=pl.BlockSpec((N_PAD // 128, 128), lambda i: (0, 0)),
      out_shape=jax.ShapeDtypeStruct((N_PAD // 128, 128), jnp.float32),
  )(d3r, d3c)


def _finish_body(p_ref, cnt_ref, xr_ref, b_ref, gam_ref, bet_ref, o_ref):
  agg = p_ref[0, :N] + p_ref[1, :N]
  cnt = cnt_ref[...]
  out = agg / jnp.maximum(cnt, 1.0) + xr_ref[...] + b_ref[...]
  mean = jnp.mean(out, axis=0, keepdims=True)
  var = jnp.mean((out - mean) ** 2, axis=0, keepdims=True)
  out = (out - mean) * jax.lax.rsqrt(var + 1e-5) * gam_ref[...] + bet_ref[...]
  o_ref[...] = jnp.where(out >= 0, out, 0.01 * out)


def _finish(partials, cnt, xr, bias, gamma, beta):
  return pl.pallas_call(
      _finish_body,
      out_shape=jax.ShapeDtypeStruct((N, D_OUT), jnp.float32),
  )(partials, cnt, xr, bias[None, :], gamma[None, :], beta[None, :])


@jax.jit
def kernel(x, edge_idx, edge_attr, g, mu, sigma, W_root, bias, gamma, beta):
  xg, xr = _input_transform(x, g, W_root)
  src = edge_idx[0].astype(jnp.int32)
  dst = edge_idx[1].astype(jnp.int32)
  zeros = jnp.zeros((N_PAD, ACC_W), jnp.float32)
  partials = _sc_aggregate(xg, src, dst, edge_attr, mu, sigma, zeros)
  cnt2d = _degree_count(dst)
  cnt = cnt2d.reshape(N_PAD)[:N].reshape(N, 1)
  return _finish(partials, cnt, xr, bias, gamma, beta)


# speedup vs baseline: 4.0910x; 1.1072x over previous
import jax
import jax.numpy as jnp
from jax import lax
from jax.experimental import pallas as pl
from jax.experimental.pallas import tpu as pltpu
from jax.experimental.pallas import tpu_sc as plsc

N = 10000
E = 320000
D_IN = 128
D_OUT = 128
K = 4
DIM = 16
EPS = 1e-15

NUM_CORES = 2
NUM_SUBCORES = 16
NUM_TILES = NUM_CORES * NUM_SUBCORES
E_PER_TILE = 10240
E_PAD = E_PER_TILE * NUM_TILES
CHUNK = 32
SUPER = 8
CPT = E_PER_TILE // CHUNK
NUM_SUPERS = CPT // SUPER
EA_ROWS_TILE = E_PER_TILE * DIM // 128
EA_ROWS_SUP = SUPER * CHUNK * DIM // 128
CNT_BLK = 512
CNT_STEPS = E // CNT_BLK
N_PAD = 10240
N_ACC = 10008
ROWS_MAIN = 632
ROWS_LAST = N_ACC - 15 * ROWS_MAIN


def _mm_body(x_ref, g_ref, wr_ref, xg_ref, xr_ref):
  x = x_ref[...]
  xg_ref[...] = jnp.dot(x, g_ref[...], preferred_element_type=jnp.float32)
  xr_ref[...] = jnp.dot(x, wr_ref[...], preferred_element_type=jnp.float32)


def _input_transform(x, g, w_root):
  return pl.pallas_call(
      _mm_body,
      out_shape=(
          jax.ShapeDtypeStruct((N, K * D_OUT), jnp.float32),
          jax.ShapeDtypeStruct((N, D_OUT), jnp.float32),
      ),
  )(x, g, w_root)


def _sc_body(xg_hbm, src_hbm, dst_hbm, ea_hbm, mu_hbm, sig_hbm, zeros_hbm,
             out_hbm, acc, src_sv, dst_sv, ea_sv, rows0, rows1, msg0, msg1,
             mu_v, sig_v, semg0, semg1, sems0, sems1):
  cid = lax.axis_index("c")
  sid = lax.axis_index("s")
  wid = sid * NUM_CORES + cid

  @pl.when(sid < 15)
  def _():
    pltpu.sync_copy(zeros_hbm.at[pl.ds(sid * ROWS_MAIN, ROWS_MAIN)],
                    acc.at[pl.ds(sid * ROWS_MAIN, ROWS_MAIN)])

  @pl.when(sid == 15)
  def _():
    pltpu.sync_copy(zeros_hbm.at[pl.ds(15 * ROWS_MAIN, ROWS_LAST)],
                    acc.at[pl.ds(15 * ROWS_MAIN, ROWS_LAST)])

  pltpu.sync_copy(mu_hbm, mu_v)
  pltpu.sync_copy(sig_hbm, sig_v)
  mus = []
  invs = []
  for k in range(K):
    m = mu_v[k]
    s = sig_v[k]
    mus.append(m)
    invs.append(0.5 / (EPS + s * s))

  lanes = lax.iota(jnp.int32, 16)
  bfly = [lax.bitwise_xor(lanes, jnp.full((16,), sh, jnp.int32))
          for sh in (8, 4, 2, 1)]

  dnums = lax.GatherDimensionNumbers(
      offset_dims=(), collapsed_slice_dims=(0,), start_index_map=(0,))

  def lane_sum(v):
    for idx in bfly:
      v = v + lax.gather(v, idx[:, None], dnums, (1,),
                         mode=lax.GatherScatterMode.PROMISE_IN_BOUNDS)
    return v

  plsc.subcore_barrier()

  rows_bufs = (rows0, rows1)
  msg_bufs = (msg0, msg1)
  semg = (semg0, semg1)
  sems = (sems0, sems1)

  def super_body(si, _):
    pltpu.sync_copy(src_hbm.at[wid, pl.ds(si * SUPER, SUPER)], src_sv)
    pltpu.sync_copy(dst_hbm.at[wid, pl.ds(si * SUPER, SUPER)], dst_sv)
    pltpu.sync_copy(
        ea_hbm.at[pl.ds(wid * EA_ROWS_TILE + si * EA_ROWS_SUP, EA_ROWS_SUP)],
        ea_sv)

    gh = {0: pltpu.async_copy(xg_hbm.at[src_sv.at[0]], rows0, semg0)}
    sh = {}
    for j in range(SUPER):
      if j + 1 < SUPER:
        gh[j + 1] = pltpu.async_copy(
            xg_hbm.at[src_sv.at[j + 1]], rows_bufs[(j + 1) % 2],
            semg[(j + 1) % 2])
      gh[j].wait()
      if j >= 2:
        sh[j - 2].wait()
      rows_v = rows_bufs[j % 2]
      msg_v = msg_bufs[j % 2]

      @plsc.parallel_loop(0, CHUNK // 8)
      def group_body(ri):
        for u in range(8):
          e = ri * 8 + u
          ea = ea_sv[j * (CHUNK // 8) + ri, pl.ds(u * 16, 16)]
          accs = [None] * (D_OUT // 16)
          for k in range(K):
            d = ea - mus[k]
            t = d * d * invs[k]
            w = jnp.exp(-lane_sum(t))
            for jj in range(D_OUT // 16):
              r = rows_v[e, pl.ds(k * D_OUT + jj * 16, 16)]
              if k == 0:
                accs[jj] = w * r
              else:
                accs[jj] = accs[jj] + w * r
          for jj in range(D_OUT // 16):
            msg_v[e, pl.ds(jj * 16, 16)] = accs[jj]

      sh[j] = pltpu.async_copy(msg_v, acc.at[dst_sv.at[j]], sems[j % 2],
                               add=True)
    sh[SUPER - 2].wait()
    sh[SUPER - 1].wait()
    return 0

  lax.fori_loop(0, NUM_SUPERS, super_body, 0)

  plsc.subcore_barrier()

  @pl.when(sid < 15)
  def _():
    pltpu.sync_copy(acc.at[pl.ds(sid * ROWS_MAIN, ROWS_MAIN)],
                    out_hbm.at[cid, pl.ds(sid * ROWS_MAIN, ROWS_MAIN)])

  @pl.when(sid == 15)
  def _():
    pltpu.sync_copy(acc.at[pl.ds(15 * ROWS_MAIN, ROWS_LAST)],
                    out_hbm.at[cid, pl.ds(15 * ROWS_MAIN, ROWS_LAST)])


def _sc_aggregate(xg, src3d, dst3d, ea2d, mu, sigma, zeros):
  mesh = plsc.VectorSubcoreMesh(
      core_axis_name="c", subcore_axis_name="s",
      num_cores=NUM_CORES, num_subcores=NUM_SUBCORES)
  run = pl.kernel(
      _sc_body,
      out_type=jax.ShapeDtypeStruct((NUM_CORES, N_ACC, D_OUT), jnp.float32),
      mesh=mesh,
      scratch_types=[
          pltpu.VMEM_SHARED((N_ACC, D_OUT), jnp.float32),
          pltpu.VMEM((SUPER, CHUNK), jnp.int32),
          pltpu.VMEM((SUPER, CHUNK), jnp.int32),
          pltpu.VMEM((EA_ROWS_SUP, 128), jnp.float32),
          pltpu.VMEM((CHUNK, K * D_OUT), jnp.float32),
          pltpu.VMEM((CHUNK, K * D_OUT), jnp.float32),
          pltpu.VMEM((CHUNK, D_OUT), jnp.float32),
          pltpu.VMEM((CHUNK, D_OUT), jnp.float32),
          pltpu.VMEM((K, DIM), jnp.float32),
          pltpu.VMEM((K, DIM), jnp.float32),
          pltpu.SemaphoreType.DMA,
          pltpu.SemaphoreType.DMA,
          pltpu.SemaphoreType.DMA,
          pltpu.SemaphoreType.DMA,
      ],
  )
  return run(xg, src3d, dst3d, ea2d, mu, sigma, zeros)


def _cnt_body(dr_ref, dc_ref, o_ref):
  i = pl.program_id(0)
  d_row = dr_ref[0]
  d_col = dc_ref[0]
  hi_row = d_row // 128
  lo_col = d_col - (d_col // 128) * 128
  ha = (lax.broadcasted_iota(jnp.int32, (N_PAD // 128, CNT_BLK), 0)
        == hi_row).astype(jnp.float32)
  hb = (lax.broadcasted_iota(jnp.int32, (CNT_BLK, 128), 1)
        == lo_col).astype(jnp.float32)

  @pl.when(i == 0)
  def _():
    o_ref[...] = jnp.zeros_like(o_ref)

  o_ref[...] += jnp.dot(ha, hb, preferred_element_type=jnp.float32)


def _degree_count(dst):
  d3r = dst.reshape(CNT_STEPS, 1, CNT_BLK)
  d3c = dst.reshape(CNT_STEPS, CNT_BLK, 1)
  return pl.pallas_call(
      _cnt_body,
      grid=(CNT_STEPS,),
      in_specs=[
          pl.BlockSpec((1, 1, CNT_BLK), lambda i: (i, 0, 0)),
          pl.BlockSpec((1, CNT_BLK, 1), lambda i: (i, 0, 0)),
      ],
      out_specs=pl.BlockSpec((N_PAD // 128, 128), lambda i: (0, 0)),
      out_shape=jax.ShapeDtypeStruct((N_PAD // 128, 128), jnp.float32),
  )(d3r, d3c)


def _finish_body(p_ref, cnt_ref, xr_ref, b_ref, gam_ref, bet_ref, o_ref):
  agg = p_ref[0, :N] + p_ref[1, :N]
  cnt = cnt_ref[...]
  out = agg / jnp.maximum(cnt, 1.0) + xr_ref[...] + b_ref[...]
  mean = jnp.mean(out, axis=0, keepdims=True)
  var = jnp.mean((out - mean) ** 2, axis=0, keepdims=True)
  out = (out - mean) * jax.lax.rsqrt(var + 1e-5) * gam_ref[...] + bet_ref[...]
  o_ref[...] = jnp.where(out >= 0, out, 0.01 * out)


def _finish(partials, cnt, xr, bias, gamma, beta):
  return pl.pallas_call(
      _finish_body,
      out_shape=jax.ShapeDtypeStruct((N, D_OUT), jnp.float32),
  )(partials, cnt, xr, bias[None, :], gamma[None, :], beta[None, :])


@jax.jit
def kernel(x, edge_idx, edge_attr, g, mu, sigma, W_root, bias, gamma, beta):
  xg, xr = _input_transform(x, g, W_root)
  n_fill = E_PAD - E
  src_p = jnp.concatenate(
      [edge_idx[0].astype(jnp.int32), jnp.zeros((n_fill,), jnp.int32)])
  dst_p = jnp.concatenate(
      [edge_idx[1].astype(jnp.int32),
       N + (jnp.arange(n_fill, dtype=jnp.int32) % 8)])
  ea_p = jnp.concatenate(
      [edge_attr, jnp.zeros((n_fill, DIM), jnp.float32)])
  src3d = src_p.reshape(NUM_TILES, CPT, CHUNK)
  dst3d = dst_p.reshape(NUM_TILES, CPT, CHUNK)
  ea2d = ea_p.reshape(NUM_TILES * EA_ROWS_TILE, 128)
  zeros = jnp.zeros((N_ACC, D_OUT), jnp.float32)
  partials = _sc_aggregate(xg, src3d, dst3d, ea2d, mu, sigma, zeros)
  cnt2d = _degree_count(edge_idx[1].astype(jnp.int32))
  cnt = cnt2d.reshape(N_PAD)[:N].reshape(N, 1)
  return _finish(partials, cnt, xr, bias, gamma, beta)


# speedup vs baseline: 5.1977x; 1.2705x over previous
import jax
import jax.numpy as jnp
from jax import lax
from jax.experimental import pallas as pl
from jax.experimental.pallas import tpu as pltpu
from jax.experimental.pallas import tpu_sc as plsc

N = 10000
E = 320000
D_IN = 128
D_OUT = 128
K = 4
DIM = 16
EPS = 1e-15

NUM_CORES = 2
NUM_SUBCORES = 16
NUM_TILES = NUM_CORES * NUM_SUBCORES
E_PER_TILE = 10240
E_PAD = E_PER_TILE * NUM_TILES
CHUNK = 32
SUPER = 8
CPT = E_PER_TILE // CHUNK
NUM_SUPERS = CPT // SUPER
EA_ROWS_TILE = E_PER_TILE * DIM // 128
EA_ROWS_SUP = SUPER * CHUNK * DIM // 128
CNT_BLK = 512
CNT_STEPS = E // CNT_BLK
N_PAD = 10240
N_ACC = 10008
ROWS_MAIN = 632
ROWS_LAST = N_ACC - 15 * ROWS_MAIN


def _mm_body(x_ref, g_ref, wr_ref, xg_ref, xr_ref):
  x = x_ref[...]
  xg_ref[...] = jnp.dot(x, g_ref[...], preferred_element_type=jnp.float32)
  xr_ref[...] = jnp.dot(x, wr_ref[...], preferred_element_type=jnp.float32)


def _input_transform(x, g, w_root):
  return pl.pallas_call(
      _mm_body,
      out_shape=(
          jax.ShapeDtypeStruct((N, K * D_OUT), jnp.float32),
          jax.ShapeDtypeStruct((N, D_OUT), jnp.float32),
      ),
  )(x, g, w_root)


def _sc_body(xg_hbm, src_hbm, dst_hbm, ea_hbm, mu_hbm, sig_hbm, zeros_hbm,
             out_hbm, acc, src_sv, dst_sv, ea_t_sv, rows0, rows1, msg0, msg1,
             mu_v, sig_v, semg0, semg1, sems0, sems1):
  cid = lax.axis_index("c")
  sid = lax.axis_index("s")
  wid = sid * NUM_CORES + cid

  @pl.when(sid < 15)
  def _():
    pltpu.sync_copy(zeros_hbm.at[pl.ds(sid * ROWS_MAIN, ROWS_MAIN)],
                    acc.at[pl.ds(sid * ROWS_MAIN, ROWS_MAIN)])

  @pl.when(sid == 15)
  def _():
    pltpu.sync_copy(zeros_hbm.at[pl.ds(15 * ROWS_MAIN, ROWS_LAST)],
                    acc.at[pl.ds(15 * ROWS_MAIN, ROWS_LAST)])

  pltpu.sync_copy(mu_hbm, mu_v)
  pltpu.sync_copy(sig_hbm, sig_v)

  dnums = lax.GatherDimensionNumbers(
      offset_dims=(), collapsed_slice_dims=(0,), start_index_map=(0,))

  def bcast(v, u):
    idx = jnp.full((16,), u, jnp.int32)
    return lax.gather(v, idx[:, None], dnums, (1,),
                      mode=lax.GatherScatterMode.PROMISE_IN_BOUNDS)

  plsc.subcore_barrier()

  rows_bufs = (rows0, rows1)
  msg_bufs = (msg0, msg1)
  semg = (semg0, semg1)
  sems = (sems0, sems1)

  def super_body(si, _):
    pltpu.sync_copy(src_hbm.at[wid, pl.ds(si * SUPER, SUPER)], src_sv)
    pltpu.sync_copy(dst_hbm.at[wid, pl.ds(si * SUPER, SUPER)], dst_sv)
    pltpu.sync_copy(ea_hbm.at[wid, si], ea_t_sv)

    gh = {0: pltpu.async_copy(xg_hbm.at[src_sv.at[0]], rows0, semg0)}
    sh = {}
    for j in range(SUPER):
      if j + 1 < SUPER:
        gh[j + 1] = pltpu.async_copy(
            xg_hbm.at[src_sv.at[j + 1]], rows_bufs[(j + 1) % 2],
            semg[(j + 1) % 2])
      gh[j].wait()
      if j >= 2:
        sh[j - 2].wait()
      rows_v = rows_bufs[j % 2]
      msg_v = msg_bufs[j % 2]

      @plsc.parallel_loop(0, CHUNK // 16)
      def group_body(gi):
        lane0 = j * CHUNK + gi * 16
        eds = [ea_t_sv[d, pl.ds(lane0, 16)] for d in range(DIM)]
        ws = []
        for k in range(K):
          terms = []
          for d in range(DIM):
            fi = k * DIM + d
            bmu = mu_v[fi // 8, pl.ds((fi % 8) * 16, 16)]
            binv = sig_v[fi // 8, pl.ds((fi % 8) * 16, 16)]
            dd = eds[d] - bmu
            terms.append(dd * dd * binv)
          while len(terms) > 1:
            terms = [terms[i] + terms[i + 1]
                     for i in range(0, len(terms), 2)]
          ws.append(jnp.exp(-terms[0]))

        @plsc.parallel_loop(0, 16)
        def edge_u(u):
          e = gi * 16 + u
          accs = [None] * (D_OUT // 16)
          for k in range(K):
            w = bcast(ws[k], u)
            for jj in range(D_OUT // 16):
              r = rows_v[e, pl.ds(k * D_OUT + jj * 16, 16)]
              if k == 0:
                accs[jj] = w * r
              else:
                accs[jj] = accs[jj] + w * r
          for jj in range(D_OUT // 16):
            msg_v[e, pl.ds(jj * 16, 16)] = accs[jj]

      sh[j] = pltpu.async_copy(msg_v, acc.at[dst_sv.at[j]], sems[j % 2],
                               add=True)
    sh[SUPER - 2].wait()
    sh[SUPER - 1].wait()
    return 0

  lax.fori_loop(0, NUM_SUPERS, super_body, 0)

  plsc.subcore_barrier()

  @pl.when(sid < 15)
  def _():
    pltpu.sync_copy(acc.at[pl.ds(sid * ROWS_MAIN, ROWS_MAIN)],
                    out_hbm.at[cid, pl.ds(sid * ROWS_MAIN, ROWS_MAIN)])

  @pl.when(sid == 15)
  def _():
    pltpu.sync_copy(acc.at[pl.ds(15 * ROWS_MAIN, ROWS_LAST)],
                    out_hbm.at[cid, pl.ds(15 * ROWS_MAIN, ROWS_LAST)])


def _sc_aggregate(xg, src3d, dst3d, ea2d, mu, sigma, zeros):
  mesh = plsc.VectorSubcoreMesh(
      core_axis_name="c", subcore_axis_name="s",
      num_cores=NUM_CORES, num_subcores=NUM_SUBCORES)
  run = pl.kernel(
      _sc_body,
      out_type=jax.ShapeDtypeStruct((NUM_CORES, N_ACC, D_OUT), jnp.float32),
      mesh=mesh,
      scratch_types=[
          pltpu.VMEM_SHARED((N_ACC, D_OUT), jnp.float32),
          pltpu.VMEM((SUPER, CHUNK), jnp.int32),
          pltpu.VMEM((SUPER, CHUNK), jnp.int32),
          pltpu.VMEM((DIM, SUPER * CHUNK), jnp.float32),
          pltpu.VMEM((CHUNK, K * D_OUT), jnp.float32),
          pltpu.VMEM((CHUNK, K * D_OUT), jnp.float32),
          pltpu.VMEM((CHUNK, D_OUT), jnp.float32),
          pltpu.VMEM((CHUNK, D_OUT), jnp.float32),
          pltpu.VMEM((8, 128), jnp.float32),
          pltpu.VMEM((8, 128), jnp.float32),
          pltpu.SemaphoreType.DMA,
          pltpu.SemaphoreType.DMA,
          pltpu.SemaphoreType.DMA,
          pltpu.SemaphoreType.DMA,
      ],
  )
  return run(xg, src3d, dst3d, ea2d, mu, sigma, zeros)


def _cnt_body(dr_ref, dc_ref, o_ref):
  i = pl.program_id(0)
  d_row = dr_ref[0]
  d_col = dc_ref[0]
  hi_row = d_row // 128
  lo_col = d_col - (d_col // 128) * 128
  ha = (lax.broadcasted_iota(jnp.int32, (N_PAD // 128, CNT_BLK), 0)
        == hi_row).astype(jnp.float32)
  hb = (lax.broadcasted_iota(jnp.int32, (CNT_BLK, 128), 1)
        == lo_col).astype(jnp.float32)

  @pl.when(i == 0)
  def _():
    o_ref[...] = jnp.zeros_like(o_ref)

  o_ref[...] += jnp.dot(ha, hb, preferred_element_type=jnp.float32)


def _degree_count(dst):
  d3r = dst.reshape(CNT_STEPS, 1, CNT_BLK)
  d3c = dst.reshape(CNT_STEPS, CNT_BLK, 1)
  return pl.pallas_call(
      _cnt_body,
      grid=(CNT_STEPS,),
      in_specs=[
          pl.BlockSpec((1, 1, CNT_BLK), lambda i: (i, 0, 0)),
          pl.BlockSpec((1, CNT_BLK, 1), lambda i: (i, 0, 0)),
      ],
      out_specs=pl.BlockSpec((N_PAD // 128, 128), lambda i: (0, 0)),
      out_shape=jax.ShapeDtypeStruct((N_PAD // 128, 128), jnp.float32),
  )(d3r, d3c)


def _finish_body(p_ref, cnt_ref, xr_ref, b_ref, gam_ref, bet_ref, o_ref):
  agg = p_ref[0, :N] + p_ref[1, :N]
  cnt = cnt_ref[...]
  out = agg / jnp.maximum(cnt, 1.0) + xr_ref[...] + b_ref[...]
  mean = jnp.mean(out, axis=0, keepdims=True)
  var = jnp.mean((out - mean) ** 2, axis=0, keepdims=True)
  out = (out - mean) * jax.lax.rsqrt(var + 1e-5) * gam_ref[...] + bet_ref[...]
  o_ref[...] = jnp.where(out >= 0, out, 0.01 * out)


def _finish(partials, cnt, xr, bias, gamma, beta):
  return pl.pallas_call(
      _finish_body,
      out_shape=jax.ShapeDtypeStruct((N, D_OUT), jnp.float32),
  )(partials, cnt, xr, bias[None, :], gamma[None, :], beta[None, :])


@jax.jit
def kernel(x, edge_idx, edge_attr, g, mu, sigma, W_root, bias, gamma, beta):
  xg, xr = _input_transform(x, g, W_root)
  n_fill = E_PAD - E
  src_p = jnp.concatenate(
      [edge_idx[0].astype(jnp.int32), jnp.zeros((n_fill,), jnp.int32)])
  dst_p = jnp.concatenate(
      [edge_idx[1].astype(jnp.int32),
       N + (jnp.arange(n_fill, dtype=jnp.int32) % 8)])
  ea_p = jnp.concatenate(
      [edge_attr, jnp.zeros((n_fill, DIM), jnp.float32)])
  src3d = src_p.reshape(NUM_TILES, CPT, CHUNK)
  dst3d = dst_p.reshape(NUM_TILES, CPT, CHUNK)
  ea_t = ea_p.reshape(NUM_TILES, NUM_SUPERS, SUPER * CHUNK, DIM)
  ea_t = ea_t.transpose(0, 1, 3, 2)
  mu_b = jnp.broadcast_to(mu[:, :, None], (K, DIM, 16)).reshape(8, 128)
  inv_b = jnp.broadcast_to(
      (0.5 / (EPS + sigma ** 2))[:, :, None], (K, DIM, 16)).reshape(8, 128)
  zeros = jnp.zeros((N_ACC, D_OUT), jnp.float32)
  partials = _sc_aggregate(xg, src3d, dst3d, ea_t, mu_b, inv_b, zeros)
  cnt2d = _degree_count(edge_idx[1].astype(jnp.int32))
  cnt = cnt2d.reshape(N_PAD)[:N].reshape(N, 1)
  return _finish(partials, cnt, xr, bias, gamma, beta)
